# Initial kernel scaffold; baseline (speedup 1.0000x reference)
import functools

import jax
import jax.numpy as jnp
from jax import lax
from jax.experimental import pallas as pl
from jax.experimental.pallas import tpu as pltpu
from jax.experimental.pallas import tpu_sc as plsc

L = 4
N = 10000
E = 320000
D = 128
DE = 4

NC = 2
NS = 16
NW = NC * NS
KE = 80
CHUNKS = E // KE
CPT = CHUNKS // NW
RPT = N // NS
ZROWS = 125

_MESH = dict(core_axis_name="c", subcore_axis_name="s", num_cores=NC,
             num_subcores=NS)


def _gather_body(a_hbm, b_hbm, dst_hbm, src_hbm, out_hbm,
                 idx_d, idx_s, rows_a, rows_b, sem_a, sem_b):
    wid = lax.axis_index("s") * NC + lax.axis_index("c")
    cbase = wid * CPT

    def chunk(j, carry):
        c = cbase + j
        pltpu.sync_copy(dst_hbm.at[c], idx_d)
        pltpu.sync_copy(src_hbm.at[c], idx_s)
        cp_a = pltpu.async_copy(a_hbm.at[idx_d], rows_a, sem_a)
        cp_b = pltpu.async_copy(b_hbm.at[idx_s], rows_b, sem_b)
        cp_a.wait()
        cp_b.wait()

        def add_e(e, carry2):
            for d8 in range(D // 16):
                sl = pl.ds(d8 * 16, 16)
                plsc.addupdate(rows_a.at[e, sl], rows_b[e, sl])
            return carry2

        lax.fori_loop(0, KE, add_e, 0)
        pltpu.sync_copy(rows_a, out_hbm.at[pl.ds(c * KE, KE)])
        return carry

    lax.fori_loop(0, CPT, chunk, 0)


_sc_gather = functools.partial(
    pl.kernel,
    _gather_body,
    out_type=jax.ShapeDtypeStruct((E, D), jnp.float32),
    mesh=plsc.VectorSubcoreMesh(**_MESH),
    scratch_types=[
        pltpu.VMEM((KE,), jnp.int32),
        pltpu.VMEM((KE,), jnp.int32),
        pltpu.VMEM((KE, D), jnp.float32),
        pltpu.VMEM((KE, D), jnp.float32),
        pltpu.SemaphoreType.DMA,
        pltpu.SemaphoreType.DMA,
    ],
)()


def _scatter_body(m2_hbm, dst_hbm, out_hbm, idx_buf, rows, zbuf, accum, sem):
    cid = lax.axis_index("c")
    sid = lax.axis_index("s")
    wid = sid * NC + cid
    zero = jnp.zeros((16,), jnp.float32)

    def zb(i, carry):
        for d8 in range(D // 16):
            zbuf[i, pl.ds(d8 * 16, 16)] = zero
        return carry

    lax.fori_loop(0, ZROWS, zb, 0)
    for k in range(RPT // ZROWS):
        pltpu.sync_copy(zbuf, accum.at[pl.ds(sid * RPT + k * ZROWS, ZROWS)])
    plsc.subcore_barrier()

    cbase = wid * CPT
    pltpu.sync_copy(dst_hbm.at[pl.ds(cbase, CPT)], idx_buf)

    def chunk(j, carry):
        pltpu.sync_copy(m2_hbm.at[pl.ds((cbase + j) * KE, KE)], rows)
        pltpu.sync_copy(rows, accum.at[idx_buf.at[j]], add=True)
        return carry

    lax.fori_loop(0, CPT, chunk, 0)
    plsc.subcore_barrier()
    for k in range(RPT // ZROWS):
        sl = pl.ds(sid * RPT + k * ZROWS, ZROWS)
        pltpu.sync_copy(accum.at[sl], out_hbm.at[cid, sl])


_sc_scatter = functools.partial(
    pl.kernel,
    _scatter_body,
    out_type=jax.ShapeDtypeStruct((NC, N, D), jnp.float32),
    mesh=plsc.VectorSubcoreMesh(**_MESH),
    scratch_types=[
        pltpu.VMEM((CPT, KE), jnp.int32),
        pltpu.VMEM((KE, D), jnp.float32),
        pltpu.VMEM((ZROWS, D), jnp.float32),
        pltpu.VMEM_SHARED((N, D), jnp.float32),
        pltpu.SemaphoreType.DMA,
    ],
)()


BE = 3200


def _msg_body(pre_ref, ea_ref, w1e_ref, c1_ref, w2_ref, c2_ref, out_ref):
    c = jnp.dot(ea_ref[...], w1e_ref[...], preferred_element_type=jnp.float32)
    m1 = jnp.maximum(pre_ref[...] + c + c1_ref[...], 0.0)
    m2 = jnp.dot(m1, w2_ref[...], preferred_element_type=jnp.float32)
    out_ref[...] = jnp.maximum(m2 + c2_ref[...], 0.0)


_tc_message = pl.pallas_call(
    _msg_body,
    grid=(E // BE,),
    in_specs=[
        pl.BlockSpec((BE, D), lambda i: (i, 0)),
        pl.BlockSpec((BE, 8), lambda i: (i, 0)),
        pl.BlockSpec((8, D), lambda i: (0, 0)),
        pl.BlockSpec((1, D), lambda i: (0, 0)),
        pl.BlockSpec((D, D), lambda i: (0, 0)),
        pl.BlockSpec((1, D), lambda i: (0, 0)),
    ],
    out_specs=pl.BlockSpec((BE, D), lambda i: (i, 0)),
    out_shape=jax.ShapeDtypeStruct((E, D), jnp.float32),
)


BN = 2000


def _upd_body(h_ref, p0_ref, p1_ref, u1a_ref, u1b_ref, d1_ref, u2_ref,
              d2_ref, wa_ref, wb_ref, h_out, a_out, b_out):
    h = h_ref[...]
    aggr = p0_ref[...] + p1_ref[...]
    u1 = jnp.dot(h, u1a_ref[...], preferred_element_type=jnp.float32)
    u1 += jnp.dot(aggr, u1b_ref[...], preferred_element_type=jnp.float32)
    u1 = jnp.maximum(u1 + d1_ref[...], 0.0)
    u2 = jnp.dot(u1, u2_ref[...], preferred_element_type=jnp.float32)
    hn = h + jnp.maximum(u2 + d2_ref[...], 0.0)
    h_out[...] = hn
    a_out[...] = jnp.dot(hn, wa_ref[...], preferred_element_type=jnp.float32)
    b_out[...] = jnp.dot(hn, wb_ref[...], preferred_element_type=jnp.float32)


def _mat_spec():
    return pl.BlockSpec((D, D), lambda i: (0, 0))


def _vec_spec():
    return pl.BlockSpec((1, D), lambda i: (0, 0))


def _node_spec():
    return pl.BlockSpec((BN, D), lambda i: (i, 0))


_tc_update = pl.pallas_call(
    _upd_body,
    grid=(N // BN,),
    in_specs=[
        _node_spec(), _node_spec(), _node_spec(),
        _mat_spec(), _mat_spec(), _vec_spec(), _mat_spec(), _vec_spec(),
        _mat_spec(), _mat_spec(),
    ],
    out_specs=(_node_spec(), _node_spec(), _node_spec()),
    out_shape=(
        jax.ShapeDtypeStruct((N, D), jnp.float32),
        jax.ShapeDtypeStruct((N, D), jnp.float32),
        jax.ShapeDtypeStruct((N, D), jnp.float32),
    ),
)


def _init_body(x_ref, w0_ref, b0_ref, wa_ref, wb_ref, h_out, a_out, b_out):
    h = jnp.dot(x_ref[...], w0_ref[...], preferred_element_type=jnp.float32)
    h += b0_ref[...]
    h_out[...] = h
    a_out[...] = jnp.dot(h, wa_ref[...], preferred_element_type=jnp.float32)
    b_out[...] = jnp.dot(h, wb_ref[...], preferred_element_type=jnp.float32)


_tc_init = pl.pallas_call(
    _init_body,
    grid=(N // BN,),
    in_specs=[
        _node_spec(),
        _mat_spec(), _vec_spec(), _mat_spec(), _mat_spec(),
    ],
    out_specs=(_node_spec(), _node_spec(), _node_spec()),
    out_shape=(
        jax.ShapeDtypeStruct((N, D), jnp.float32),
        jax.ShapeDtypeStruct((N, D), jnp.float32),
        jax.ShapeDtypeStruct((N, D), jnp.float32),
    ),
)


def kernel(x, edge_index, edge_attr, W0, b0, mW1, mb1, mW2, mb2, uW1, ub1,
           uW2, ub2, mg1, mg2, ug1, ug2, mv1, mv2, uv1, uv2, mB1, mB2, uB1,
           uB2, mm1, mm2, um1, um2):
    eps = 1e-5
    s1 = mg1 * lax.rsqrt(mv1 + eps)
    W1f = mW1 * s1[:, None, :]
    c1 = mb1 * s1 + mB1 - mm1 * s1
    s2 = mg2 * lax.rsqrt(mv2 + eps)
    W2f = mW2 * s2[:, None, :]
    c2 = mb2 * s2 + mB2 - mm2 * s2
    t1 = ug1 * lax.rsqrt(uv1 + eps)
    U1f = uW1 * t1[:, None, :]
    d1 = ub1 * t1 + uB1 - um1 * t1
    t2 = ug2 * lax.rsqrt(uv2 + eps)
    U2f = uW2 * t2[:, None, :]
    d2 = ub2 * t2 + uB2 - um2 * t2

    W1a = W1f[:, :D, :]
    W1b = W1f[:, D:2 * D, :]
    W1e = jnp.pad(W1f[:, 2 * D:, :], ((0, 0), (0, 8 - DE), (0, 0)))
    U1a = U1f[:, :D, :]
    U1b = U1f[:, D:2 * D, :]

    ea8 = jnp.pad(edge_attr, ((0, 0), (0, 8 - DE)))
    src2d = edge_index[0].reshape(CHUNKS, KE)
    dst2d = edge_index[1].reshape(CHUNKS, KE)

    h, A, B = _tc_init(x, W0, b0.reshape(1, D), W1a[0], W1b[0])
    for l in range(L):
        pre = _sc_gather(A, B, dst2d, src2d)
        m2 = _tc_message(pre, ea8, W1e[l], c1[l].reshape(1, D), W2f[l],
                         c2[l].reshape(1, D))
        partials = _sc_scatter(m2, dst2d)
        ln = (l + 1) % L
        h, A, B = _tc_update(h, partials[0], partials[1], U1a[l], U1b[l],
                             d1[l].reshape(1, D), U2f[l], d2[l].reshape(1, D),
                             W1a[ln], W1b[ln])
    return h


# speedup vs baseline: 1.6999x; 1.6999x over previous
import functools

import jax
import jax.numpy as jnp
from jax import lax
from jax.experimental import pallas as pl
from jax.experimental.pallas import tpu as pltpu
from jax.experimental.pallas import tpu_sc as plsc

L = 4
N = 10000
E = 320000
D = 128
DE = 4

NC = 2
NS = 16
NW = NC * NS
KE = 80
CHUNKS = E // KE
CPT = CHUNKS // NW

_MESH = dict(core_axis_name="c", subcore_axis_name="s", num_cores=NC,
             num_subcores=NS)


def _gather_body(a_hbm, b_hbm, dst_hbm, src_hbm, out_hbm,
                 idx_d, idx_s, rows_a, rows_b, sem_a, sem_b):
    wid = lax.axis_index("s") * NC + lax.axis_index("c")
    cbase = wid * CPT

    def chunk(j, carry):
        c = cbase + j
        pltpu.sync_copy(dst_hbm.at[pl.ds(c * KE, KE)], idx_d)
        pltpu.sync_copy(src_hbm.at[pl.ds(c * KE, KE)], idx_s)
        cp_a = pltpu.async_copy(a_hbm.at[idx_d], rows_a, sem_a)
        cp_b = pltpu.async_copy(b_hbm.at[idx_s], rows_b, sem_b)
        cp_a.wait()
        cp_b.wait()

        def add_e(e, carry2):
            for d8 in range(D // 16):
                sl = pl.ds(d8 * 16, 16)
                plsc.addupdate(rows_a.at[e, sl], rows_b[e, sl])
            return carry2

        lax.fori_loop(0, KE, add_e, 0)
        pltpu.sync_copy(rows_a, out_hbm.at[pl.ds(c * KE, KE)])
        return carry

    lax.fori_loop(0, CPT, chunk, 0)


_sc_gather = functools.partial(
    pl.kernel,
    _gather_body,
    out_type=jax.ShapeDtypeStruct((E, D), jnp.float32),
    mesh=plsc.VectorSubcoreMesh(**_MESH),
    scratch_types=[
        pltpu.VMEM((KE,), jnp.int32),
        pltpu.VMEM((KE,), jnp.int32),
        pltpu.VMEM((KE, D), jnp.float32),
        pltpu.VMEM((KE, D), jnp.float32),
        pltpu.SemaphoreType.DMA,
        pltpu.SemaphoreType.DMA,
    ],
)()


DH = D // 2
CPT2 = CHUNKS // NS


def _scatter_body(zeros_hbm, m2_hbm, dst_hbm, out_hbm, idx_buf, rows, accum,
                  sem):
    cid = lax.axis_index("c")
    sid = lax.axis_index("s")

    @pl.when(sid == 0)
    def _zero():
        pltpu.sync_copy(zeros_hbm, accum)

    plsc.subcore_barrier()

    cbase = sid * CPT2

    def chunk(j, carry):
        off = (cbase + j) * KE
        pltpu.sync_copy(dst_hbm.at[pl.ds(off, KE)], idx_buf)
        pltpu.sync_copy(m2_hbm.at[cid, pl.ds(off, KE)], rows)
        pltpu.sync_copy(rows, accum.at[idx_buf], add=True)
        return carry

    lax.fori_loop(0, CPT2, chunk, 0)
    plsc.subcore_barrier()

    @pl.when(sid == 0)
    def _out():
        pltpu.sync_copy(accum, out_hbm.at[cid])


_sc_scatter = functools.partial(
    pl.kernel,
    _scatter_body,
    out_type=jax.ShapeDtypeStruct((NC, N, DH), jnp.float32),
    mesh=plsc.VectorSubcoreMesh(**_MESH),
    scratch_types=[
        pltpu.VMEM((KE,), jnp.int32),
        pltpu.VMEM((KE, DH), jnp.float32),
        pltpu.VMEM_SHARED((N, DH), jnp.float32),
        pltpu.SemaphoreType.DMA,
    ],
    compiler_params=pltpu.CompilerParams(use_tc_tiling_on_sc=False),
)()


BE = 3200


def _msg_body(pre_ref, ea_ref, w1e_ref, c1_ref, w2_ref, c2_ref, out_ref):
    c = jnp.dot(ea_ref[...], w1e_ref[...], preferred_element_type=jnp.float32)
    m1 = jnp.maximum(pre_ref[...] + c + c1_ref[...], 0.0)
    m2 = jnp.dot(m1, w2_ref[...], preferred_element_type=jnp.float32)
    m2 = jnp.maximum(m2 + c2_ref[...], 0.0)
    out_ref[...] = jnp.stack([m2[:, :DH], m2[:, DH:]])


_tc_message = pl.pallas_call(
    _msg_body,
    grid=(E // BE,),
    in_specs=[
        pl.BlockSpec((BE, D), lambda i: (i, 0)),
        pl.BlockSpec((BE, 8), lambda i: (i, 0)),
        pl.BlockSpec((8, D), lambda i: (0, 0)),
        pl.BlockSpec((1, D), lambda i: (0, 0)),
        pl.BlockSpec((D, D), lambda i: (0, 0)),
        pl.BlockSpec((1, D), lambda i: (0, 0)),
    ],
    out_specs=pl.BlockSpec((NC, BE, DH), lambda i: (0, i, 0)),
    out_shape=jax.ShapeDtypeStruct((NC, E, DH), jnp.float32),
)


BN = 2000


def _upd_body(h_ref, p0_ref, p1_ref, u1a_ref, u1b_ref, d1_ref, u2_ref,
              d2_ref, wa_ref, wb_ref, h_out, a_out, b_out):
    h = h_ref[...]
    aggr = jnp.concatenate([p0_ref[0], p1_ref[0]], axis=-1)
    u1 = jnp.dot(h, u1a_ref[...], preferred_element_type=jnp.float32)
    u1 += jnp.dot(aggr, u1b_ref[...], preferred_element_type=jnp.float32)
    u1 = jnp.maximum(u1 + d1_ref[...], 0.0)
    u2 = jnp.dot(u1, u2_ref[...], preferred_element_type=jnp.float32)
    hn = h + jnp.maximum(u2 + d2_ref[...], 0.0)
    h_out[...] = hn
    a_out[...] = jnp.dot(hn, wa_ref[...], preferred_element_type=jnp.float32)
    b_out[...] = jnp.dot(hn, wb_ref[...], preferred_element_type=jnp.float32)


def _mat_spec():
    return pl.BlockSpec((D, D), lambda i: (0, 0))


def _vec_spec():
    return pl.BlockSpec((1, D), lambda i: (0, 0))


def _node_spec():
    return pl.BlockSpec((BN, D), lambda i: (i, 0))


def _half_spec(c):
    return pl.BlockSpec((1, BN, DH), lambda i, c=c: (c, i, 0))


_tc_update = pl.pallas_call(
    _upd_body,
    grid=(N // BN,),
    in_specs=[
        _node_spec(), _half_spec(0), _half_spec(1),
        _mat_spec(), _mat_spec(), _vec_spec(), _mat_spec(), _vec_spec(),
        _mat_spec(), _mat_spec(),
    ],
    out_specs=(_node_spec(), _node_spec(), _node_spec()),
    out_shape=(
        jax.ShapeDtypeStruct((N, D), jnp.float32),
        jax.ShapeDtypeStruct((N, D), jnp.float32),
        jax.ShapeDtypeStruct((N, D), jnp.float32),
    ),
)


def _init_body(x_ref, w0_ref, b0_ref, wa_ref, wb_ref, h_out, a_out, b_out):
    h = jnp.dot(x_ref[...], w0_ref[...], preferred_element_type=jnp.float32)
    h += b0_ref[...]
    h_out[...] = h
    a_out[...] = jnp.dot(h, wa_ref[...], preferred_element_type=jnp.float32)
    b_out[...] = jnp.dot(h, wb_ref[...], preferred_element_type=jnp.float32)


_tc_init = pl.pallas_call(
    _init_body,
    grid=(N // BN,),
    in_specs=[
        _node_spec(),
        _mat_spec(), _vec_spec(), _mat_spec(), _mat_spec(),
    ],
    out_specs=(_node_spec(), _node_spec(), _node_spec()),
    out_shape=(
        jax.ShapeDtypeStruct((N, D), jnp.float32),
        jax.ShapeDtypeStruct((N, D), jnp.float32),
        jax.ShapeDtypeStruct((N, D), jnp.float32),
    ),
)


def kernel(x, edge_index, edge_attr, W0, b0, mW1, mb1, mW2, mb2, uW1, ub1,
           uW2, ub2, mg1, mg2, ug1, ug2, mv1, mv2, uv1, uv2, mB1, mB2, uB1,
           uB2, mm1, mm2, um1, um2):
    eps = 1e-5
    s1 = mg1 * lax.rsqrt(mv1 + eps)
    W1f = mW1 * s1[:, None, :]
    c1 = mb1 * s1 + mB1 - mm1 * s1
    s2 = mg2 * lax.rsqrt(mv2 + eps)
    W2f = mW2 * s2[:, None, :]
    c2 = mb2 * s2 + mB2 - mm2 * s2
    t1 = ug1 * lax.rsqrt(uv1 + eps)
    U1f = uW1 * t1[:, None, :]
    d1 = ub1 * t1 + uB1 - um1 * t1
    t2 = ug2 * lax.rsqrt(uv2 + eps)
    U2f = uW2 * t2[:, None, :]
    d2 = ub2 * t2 + uB2 - um2 * t2

    W1a = W1f[:, :D, :]
    W1b = W1f[:, D:2 * D, :]
    W1e = jnp.pad(W1f[:, 2 * D:, :], ((0, 0), (0, 8 - DE), (0, 0)))
    U1a = U1f[:, :D, :]
    U1b = U1f[:, D:2 * D, :]

    ea8 = jnp.pad(edge_attr, ((0, 0), (0, 8 - DE)))
    src = edge_index[0]
    dst = edge_index[1]
    zeros_nd = jnp.zeros((N, DH), jnp.float32)

    h, A, B = _tc_init(x, W0, b0.reshape(1, D), W1a[0], W1b[0])
    for l in range(L):
        pre = _sc_gather(A, B, dst, src)
        m2 = _tc_message(pre, ea8, W1e[l], c1[l].reshape(1, D), W2f[l],
                         c2[l].reshape(1, D))
        partials = _sc_scatter(zeros_nd, m2, dst)
        ln = (l + 1) % L
        h, A, B = _tc_update(h, partials, partials, U1a[l], U1b[l],
                             d1[l].reshape(1, D), U2f[l], d2[l].reshape(1, D),
                             W1a[ln], W1b[ln])
    return h


# speedup vs baseline: 2.6955x; 1.5857x over previous
import functools

import jax
import jax.numpy as jnp
from jax import lax
from jax.experimental import pallas as pl
from jax.experimental.pallas import tpu as pltpu
from jax.experimental.pallas import tpu_sc as plsc

L = 4
N = 10000
E = 320000
D = 128
DE = 4

NC = 2
NS = 16
NW = NC * NS
KE = 80
CHUNKS = E // KE
CPT = CHUNKS // NW

_MESH = dict(core_axis_name="c", subcore_axis_name="s", num_cores=NC,
             num_subcores=NS)


EPT = E // NW


def _gather_body(a_hbm, b_hbm, dst_hbm, src_hbm, out_hbm,
                 idx_d, idx_s, ra0, ra1, rb0, rb1, os0, os1,
                 sa0, sa1, sb0, sb1, so0, so1):
    wid = lax.axis_index("s") * NC + lax.axis_index("c")
    ebase = wid * EPT
    bufs = [(ra0, rb0, os0, sa0, sb0, so0), (ra1, rb1, os1, sa1, sb1, so1)]

    pltpu.sync_copy(dst_hbm.at[pl.ds(ebase, EPT)], idx_d)
    pltpu.sync_copy(src_hbm.at[pl.ds(ebase, EPT)], idx_s)

    def fire(jj, b):
        ra, rb, _, sa, sb, _ = bufs[b]
        sl = pl.ds(jj * KE, KE)
        pltpu.async_copy(a_hbm.at[idx_d.at[sl]], ra, sa)
        pltpu.async_copy(b_hbm.at[idx_s.at[sl]], rb, sb)

    def wait_gather(jj, b):
        ra, rb, _, sa, sb, _ = bufs[b]
        sl = pl.ds(jj * KE, KE)
        pltpu.make_async_copy(a_hbm.at[idx_d.at[sl]], ra, sa).wait()
        pltpu.make_async_copy(b_hbm.at[idx_s.at[sl]], rb, sb).wait()

    def add(b):
        ra, rb, os_, _, _, _ = bufs[b]

        def body(e, carry):
            for d8 in range(D // 16):
                sl = pl.ds(d8 * 16, 16)
                os_[e, sl] = ra[e, sl] + rb[e, sl]
            return carry

        lax.fori_loop(0, KE, body, 0)

    def outsl(jj):
        return out_hbm.at[pl.ds(ebase + jj * KE, KE)]

    fire(0, 0)
    fire(1, 1)

    def pair(p, carry):
        for b in range(2):
            jj = 2 * p + b
            _, _, os_, _, _, so = bufs[b]
            wait_gather(jj, b)

            @pl.when(jj >= 2)
            def _wout():
                pltpu.make_async_copy(os_, outsl(jj - 2), so).wait()

            add(b)
            pltpu.async_copy(os_, outsl(jj), so)

            @pl.when(jj + 2 < CPT)
            def _pref():
                fire(jj + 2, b)
        return carry

    lax.fori_loop(0, (CPT - 1) // 2, pair, 0)
    jj = CPT - 1
    _, _, osl0, _, _, sol0 = bufs[0]
    wait_gather(jj, 0)
    pltpu.make_async_copy(osl0, outsl(jj - 2), sol0).wait()
    add(0)
    pltpu.sync_copy(osl0, outsl(jj))
    _, _, osl1, _, _, sol1 = bufs[1]
    pltpu.make_async_copy(osl1, outsl(CPT - 2), sol1).wait()


_sc_gather = functools.partial(
    pl.kernel,
    _gather_body,
    out_type=jax.ShapeDtypeStruct((E, D), jnp.float32),
    mesh=plsc.VectorSubcoreMesh(**_MESH),
    scratch_types=(
        [pltpu.VMEM((EPT,), jnp.int32)] * 2
        + [pltpu.VMEM((KE, D), jnp.float32)] * 6
        + [pltpu.SemaphoreType.DMA] * 6
    ),
)()


DH = D // 2
CPT2 = CHUNKS // NS


EPT2 = E // NS


def _scatter_body(zeros_hbm, m2_hbm, dst_hbm, out_hbm,
                  idx_all, r0, r1, accum, s0, s1, c0, c1):
    cid = lax.axis_index("c")
    sid = lax.axis_index("s")

    @pl.when(sid == 0)
    def _zero():
        pltpu.sync_copy(zeros_hbm, accum)

    plsc.subcore_barrier()

    ebase = sid * EPT2
    pltpu.sync_copy(dst_hbm.at[pl.ds(ebase, EPT2)], idx_all)
    bufs = [(r0, s0, c0), (r1, s1, c1)]

    def m2sl(jj):
        return m2_hbm.at[cid, pl.ds(ebase + jj * KE, KE)]

    def idxsl(jj):
        return idx_all.at[pl.ds(jj * KE, KE)]

    def fire_load(jj, b):
        r, s, _ = bufs[b]
        pltpu.async_copy(m2sl(jj), r, s)

    def wait_load(jj, b):
        r, s, _ = bufs[b]
        pltpu.make_async_copy(m2sl(jj), r, s).wait()

    fire_load(0, 0)
    fire_load(1, 1)

    def pair(p, carry):
        for b in range(2):
            jj = 2 * p + b
            r, s, c = bufs[b]
            wait_load(jj, b)

            @pl.when(jj >= 2)
            def _wsc():
                pltpu.make_async_copy(r, accum.at[idxsl(jj - 2)], c).wait()

            pltpu.async_copy(r, accum.at[idxsl(jj)], c, add=True)

            @pl.when(jj + 2 < CPT2)
            def _pref():
                fire_load(jj + 2, b)
        return carry

    lax.fori_loop(0, CPT2 // 2, pair, 0)
    for b in range(2):
        r, s, c = bufs[b]
        pltpu.make_async_copy(r, accum.at[idxsl(CPT2 - 2 + b)], c).wait()

    plsc.subcore_barrier()

    @pl.when(sid == 0)
    def _out():
        pltpu.sync_copy(accum, out_hbm.at[cid])


_sc_scatter = functools.partial(
    pl.kernel,
    _scatter_body,
    out_type=jax.ShapeDtypeStruct((NC, N, DH), jnp.float32),
    mesh=plsc.VectorSubcoreMesh(**_MESH),
    scratch_types=(
        [pltpu.VMEM((EPT2,), jnp.int32)]
        + [pltpu.VMEM((KE, DH), jnp.float32)] * 2
        + [pltpu.VMEM_SHARED((N, DH), jnp.float32)]
        + [pltpu.SemaphoreType.DMA] * 4
    ),
    compiler_params=pltpu.CompilerParams(use_tc_tiling_on_sc=False),
)()


BE = 3200


def _msg_body(pre_ref, ea_ref, w1e_ref, c1_ref, w2_ref, c2_ref, out_ref):
    c = jnp.dot(ea_ref[...], w1e_ref[...], preferred_element_type=jnp.float32)
    m1 = jnp.maximum(pre_ref[...] + c + c1_ref[...], 0.0)
    m2 = jnp.dot(m1, w2_ref[...], preferred_element_type=jnp.float32)
    m2 = jnp.maximum(m2 + c2_ref[...], 0.0)
    out_ref[...] = jnp.stack([m2[:, :DH], m2[:, DH:]])


_tc_message = pl.pallas_call(
    _msg_body,
    grid=(E // BE,),
    in_specs=[
        pl.BlockSpec((BE, D), lambda i: (i, 0)),
        pl.BlockSpec((BE, 8), lambda i: (i, 0)),
        pl.BlockSpec((8, D), lambda i: (0, 0)),
        pl.BlockSpec((1, D), lambda i: (0, 0)),
        pl.BlockSpec((D, D), lambda i: (0, 0)),
        pl.BlockSpec((1, D), lambda i: (0, 0)),
    ],
    out_specs=pl.BlockSpec((NC, BE, DH), lambda i: (0, i, 0)),
    out_shape=jax.ShapeDtypeStruct((NC, E, DH), jnp.float32),
)


BN = 2000


def _upd_body(h_ref, p0_ref, p1_ref, u1a_ref, u1b_ref, d1_ref, u2_ref,
              d2_ref, wa_ref, wb_ref, h_out, a_out, b_out):
    h = h_ref[...]
    aggr = jnp.concatenate([p0_ref[0], p1_ref[0]], axis=-1)
    u1 = jnp.dot(h, u1a_ref[...], preferred_element_type=jnp.float32)
    u1 += jnp.dot(aggr, u1b_ref[...], preferred_element_type=jnp.float32)
    u1 = jnp.maximum(u1 + d1_ref[...], 0.0)
    u2 = jnp.dot(u1, u2_ref[...], preferred_element_type=jnp.float32)
    hn = h + jnp.maximum(u2 + d2_ref[...], 0.0)
    h_out[...] = hn
    a_out[...] = jnp.dot(hn, wa_ref[...], preferred_element_type=jnp.float32)
    b_out[...] = jnp.dot(hn, wb_ref[...], preferred_element_type=jnp.float32)


def _mat_spec():
    return pl.BlockSpec((D, D), lambda i: (0, 0))


def _vec_spec():
    return pl.BlockSpec((1, D), lambda i: (0, 0))


def _node_spec():
    return pl.BlockSpec((BN, D), lambda i: (i, 0))


def _half_spec(c):
    return pl.BlockSpec((1, BN, DH), lambda i, c=c: (c, i, 0))


_tc_update = pl.pallas_call(
    _upd_body,
    grid=(N // BN,),
    in_specs=[
        _node_spec(), _half_spec(0), _half_spec(1),
        _mat_spec(), _mat_spec(), _vec_spec(), _mat_spec(), _vec_spec(),
        _mat_spec(), _mat_spec(),
    ],
    out_specs=(_node_spec(), _node_spec(), _node_spec()),
    out_shape=(
        jax.ShapeDtypeStruct((N, D), jnp.float32),
        jax.ShapeDtypeStruct((N, D), jnp.float32),
        jax.ShapeDtypeStruct((N, D), jnp.float32),
    ),
)


def _init_body(x_ref, w0_ref, b0_ref, wa_ref, wb_ref, h_out, a_out, b_out):
    h = jnp.dot(x_ref[...], w0_ref[...], preferred_element_type=jnp.float32)
    h += b0_ref[...]
    h_out[...] = h
    a_out[...] = jnp.dot(h, wa_ref[...], preferred_element_type=jnp.float32)
    b_out[...] = jnp.dot(h, wb_ref[...], preferred_element_type=jnp.float32)


_tc_init = pl.pallas_call(
    _init_body,
    grid=(N // BN,),
    in_specs=[
        _node_spec(),
        _mat_spec(), _vec_spec(), _mat_spec(), _mat_spec(),
    ],
    out_specs=(_node_spec(), _node_spec(), _node_spec()),
    out_shape=(
        jax.ShapeDtypeStruct((N, D), jnp.float32),
        jax.ShapeDtypeStruct((N, D), jnp.float32),
        jax.ShapeDtypeStruct((N, D), jnp.float32),
    ),
)


def kernel(x, edge_index, edge_attr, W0, b0, mW1, mb1, mW2, mb2, uW1, ub1,
           uW2, ub2, mg1, mg2, ug1, ug2, mv1, mv2, uv1, uv2, mB1, mB2, uB1,
           uB2, mm1, mm2, um1, um2):
    eps = 1e-5
    s1 = mg1 * lax.rsqrt(mv1 + eps)
    W1f = mW1 * s1[:, None, :]
    c1 = mb1 * s1 + mB1 - mm1 * s1
    s2 = mg2 * lax.rsqrt(mv2 + eps)
    W2f = mW2 * s2[:, None, :]
    c2 = mb2 * s2 + mB2 - mm2 * s2
    t1 = ug1 * lax.rsqrt(uv1 + eps)
    U1f = uW1 * t1[:, None, :]
    d1 = ub1 * t1 + uB1 - um1 * t1
    t2 = ug2 * lax.rsqrt(uv2 + eps)
    U2f = uW2 * t2[:, None, :]
    d2 = ub2 * t2 + uB2 - um2 * t2

    W1a = W1f[:, :D, :]
    W1b = W1f[:, D:2 * D, :]
    W1e = jnp.pad(W1f[:, 2 * D:, :], ((0, 0), (0, 8 - DE), (0, 0)))
    U1a = U1f[:, :D, :]
    U1b = U1f[:, D:2 * D, :]

    ea8 = jnp.pad(edge_attr, ((0, 0), (0, 8 - DE)))
    src = edge_index[0]
    dst = edge_index[1]
    zeros_nd = jnp.zeros((N, DH), jnp.float32)

    h, A, B = _tc_init(x, W0, b0.reshape(1, D), W1a[0], W1b[0])
    for l in range(L):
        pre = _sc_gather(A, B, dst, src)
        m2 = _tc_message(pre, ea8, W1e[l], c1[l].reshape(1, D), W2f[l],
                         c2[l].reshape(1, D))
        partials = _sc_scatter(zeros_nd, m2, dst)
        ln = (l + 1) % L
        h, A, B = _tc_update(h, partials, partials, U1a[l], U1b[l],
                             d1[l].reshape(1, D), U2f[l], d2[l].reshape(1, D),
                             W1a[ln], W1b[ln])
    return h
